# Initial kernel scaffold; baseline (speedup 1.0000x reference)
import jax
import jax.numpy as jnp
from jax.experimental import pallas as pl


def kernel(xin, W, b):
    raise NotImplementedError("write your pallas kernel here")


# speedup vs baseline: 1.1056x; 1.1056x over previous
import jax
import jax.numpy as jnp
from jax.experimental import pallas as pl

_STRIDE = 32.0
_G = 19
_NA = 3
_NC = 85
_LANES = 128
_AW = (116.0, 156.0, 373.0)
_AH = (90.0, 198.0, 326.0)


def _yolo_body(x_ref, w_ref, b_ref, o_ref):
    x = x_ref[0]
    w = w_ref[...]
    mm = jax.lax.dot_general(
        x, w, (((0,), (0,)), ((), ())),
        preferred_element_type=jnp.float32,
        precision=jax.lax.Precision.HIGHEST)
    mm = mm + b_ref[...]
    lane = jax.lax.broadcasted_iota(jnp.int32, mm.shape, 1)
    row = jax.lax.broadcasted_iota(jnp.int32, mm.shape, 0)
    a = lane // _LANES
    ch = lane - a * _LANES
    gy = (row // _G).astype(jnp.float32)
    gx = (row % _G).astype(jnp.float32)
    sig = jax.nn.sigmoid(mm)
    ex = jnp.exp(mm)
    aw = jnp.where(a == 0, _AW[0], jnp.where(a == 1, _AW[1], _AW[2]))
    ah = jnp.where(a == 0, _AH[0], jnp.where(a == 1, _AH[1], _AH[2]))
    out = jnp.where(ch == 0, (sig + gx) * _STRIDE,
          jnp.where(ch == 1, (sig + gy) * _STRIDE,
          jnp.where(ch == 2, ex * aw,
          jnp.where(ch == 3, ex * ah, sig))))
    for i in range(_NA):
        o_ref[0, i] = out[:, i * _LANES:i * _LANES + _NC]


def kernel(xin, W, b):
    B, C, G, _ = xin.shape
    GG = G * G
    x3 = xin.reshape(B, C, GG)
    w2 = W.reshape(_NA, _NC, C)
    w2 = jnp.pad(w2, ((0, 0), (0, _LANES - _NC), (0, 0)))
    wp = w2.reshape(_NA * _LANES, C).T
    bp = jnp.pad(b.reshape(_NA, _NC), ((0, 0), (0, _LANES - _NC)))
    bp = bp.reshape(1, _NA * _LANES)
    out = pl.pallas_call(
        _yolo_body,
        grid=(B,),
        in_specs=[
            pl.BlockSpec((1, C, GG), lambda i: (i, 0, 0)),
            pl.BlockSpec((C, _NA * _LANES), lambda i: (0, 0)),
            pl.BlockSpec((1, _NA * _LANES), lambda i: (0, 0)),
        ],
        out_specs=pl.BlockSpec((1, _NA, GG, _NC), lambda i: (i, 0, 0, 0)),
        out_shape=jax.ShapeDtypeStruct((B, _NA, GG, _NC), jnp.float32),
    )(x3, wp, bp)
    return out.reshape(B, _NA * GG, _NC)


# speedup vs baseline: 1.5003x; 1.3571x over previous
import jax
import jax.numpy as jnp
from jax.experimental import pallas as pl

_STRIDE = 32.0
_G = 19
_NA = 3
_NC = 85
_LANES = 128
_AW = (116.0, 156.0, 373.0)
_AH = (90.0, 198.0, 326.0)


def _yolo_body(x_ref, w_ref, b_ref, o_ref):
    x = x_ref[0]
    w = w_ref[...]
    mm = jax.lax.dot_general(
        x, w, (((0,), (0,)), ((), ())),
        preferred_element_type=jnp.float32,
        precision=jax.lax.Precision.DEFAULT)
    mm = mm + b_ref[...]
    lane = jax.lax.broadcasted_iota(jnp.int32, mm.shape, 1)
    row = jax.lax.broadcasted_iota(jnp.int32, mm.shape, 0)
    a = lane // _LANES
    ch = lane - a * _LANES
    gy = (row // _G).astype(jnp.float32)
    gx = (row % _G).astype(jnp.float32)
    sig = jax.nn.sigmoid(mm)
    ex = jnp.exp(mm)
    aw = jnp.where(a == 0, _AW[0], jnp.where(a == 1, _AW[1], _AW[2]))
    ah = jnp.where(a == 0, _AH[0], jnp.where(a == 1, _AH[1], _AH[2]))
    out = jnp.where(ch == 0, (sig + gx) * _STRIDE,
          jnp.where(ch == 1, (sig + gy) * _STRIDE,
          jnp.where(ch == 2, ex * aw,
          jnp.where(ch == 3, ex * ah, sig))))
    for i in range(_NA):
        o_ref[0, i] = out[:, i * _LANES:i * _LANES + _NC]


def kernel(xin, W, b):
    B, C, G, _ = xin.shape
    GG = G * G
    x3 = xin.reshape(B, C, GG)
    w2 = W.reshape(_NA, _NC, C)
    w2 = jnp.pad(w2, ((0, 0), (0, _LANES - _NC), (0, 0)))
    wp = w2.reshape(_NA * _LANES, C).T
    bp = jnp.pad(b.reshape(_NA, _NC), ((0, 0), (0, _LANES - _NC)))
    bp = bp.reshape(1, _NA * _LANES)
    out = pl.pallas_call(
        _yolo_body,
        grid=(B,),
        in_specs=[
            pl.BlockSpec((1, C, GG), lambda i: (i, 0, 0)),
            pl.BlockSpec((C, _NA * _LANES), lambda i: (0, 0)),
            pl.BlockSpec((1, _NA * _LANES), lambda i: (0, 0)),
        ],
        out_specs=pl.BlockSpec((1, _NA, GG, _NC), lambda i: (i, 0, 0, 0)),
        out_shape=jax.ShapeDtypeStruct((B, _NA, GG, _NC), jnp.float32),
    )(x3, wp, bp)
    return out.reshape(B, _NA * GG, _NC)


# speedup vs baseline: 1.5145x; 1.0094x over previous
import jax
import jax.numpy as jnp
from jax.experimental import pallas as pl

_STRIDE = 32.0
_G = 19
_NA = 3
_NC = 85
_LANES = 128
_AW = (116.0, 156.0, 373.0)
_AH = (90.0, 198.0, 326.0)


def _yolo_body(x_ref, w_ref, b_ref, o_ref):
    x = x_ref[0]
    w = w_ref[...]
    mm_t = jax.lax.dot_general(
        w, x, (((1,), (0,)), ((), ())),
        preferred_element_type=jnp.float32,
        precision=jax.lax.Precision.DEFAULT)
    mm = jnp.transpose(mm_t) + b_ref[...]
    lane = jax.lax.broadcasted_iota(jnp.int32, mm.shape, 1)
    row = jax.lax.broadcasted_iota(jnp.int32, mm.shape, 0)
    a = lane // _LANES
    ch = lane - a * _LANES
    gy = (row // _G).astype(jnp.float32)
    gx = (row % _G).astype(jnp.float32)
    sig = jax.nn.sigmoid(mm)
    ex = jnp.exp(mm)
    aw = jnp.where(a == 0, _AW[0], jnp.where(a == 1, _AW[1], _AW[2]))
    ah = jnp.where(a == 0, _AH[0], jnp.where(a == 1, _AH[1], _AH[2]))
    out = jnp.where(ch == 0, (sig + gx) * _STRIDE,
          jnp.where(ch == 1, (sig + gy) * _STRIDE,
          jnp.where(ch == 2, ex * aw,
          jnp.where(ch == 3, ex * ah, sig))))
    for i in range(_NA):
        o_ref[0, i] = out[:, i * _LANES:i * _LANES + _NC]


def kernel(xin, W, b):
    B, C, G, _ = xin.shape
    GG = G * G
    x3 = xin.reshape(B, C, GG)
    w2 = W.reshape(_NA, _NC, C)
    w2 = jnp.pad(w2, ((0, 0), (0, _LANES - _NC), (0, 0)))
    wp = w2.reshape(_NA * _LANES, C)
    bp = jnp.pad(b.reshape(_NA, _NC), ((0, 0), (0, _LANES - _NC)))
    bp = bp.reshape(1, _NA * _LANES)
    out = pl.pallas_call(
        _yolo_body,
        grid=(B,),
        in_specs=[
            pl.BlockSpec((1, C, GG), lambda i: (i, 0, 0)),
            pl.BlockSpec((_NA * _LANES, C), lambda i: (0, 0)),
            pl.BlockSpec((1, _NA * _LANES), lambda i: (0, 0)),
        ],
        out_specs=pl.BlockSpec((1, _NA, GG, _NC), lambda i: (i, 0, 0, 0)),
        out_shape=jax.ShapeDtypeStruct((B, _NA, GG, _NC), jnp.float32),
    )(x3, wp, bp)
    return out.reshape(B, _NA * GG, _NC)


# speedup vs baseline: 1.9032x; 1.2567x over previous
import jax
import jax.numpy as jnp
from jax.experimental import pallas as pl

_STRIDE = 32.0
_G = 19
_NA = 3
_NC = 85
_LANES = 128
_AW = (116.0, 156.0, 373.0)
_AH = (90.0, 198.0, 326.0)


def _yolo_body(x_ref, w_ref, b_ref, o_ref):
    x = x_ref[0]
    w = w_ref[...]
    mm_t = jax.lax.dot_general(
        w, x, (((1,), (0,)), ((), ())),
        preferred_element_type=jnp.float32,
        precision=jax.lax.Precision.DEFAULT)
    mm = jnp.transpose(mm_t) + b_ref[...]
    lane = jax.lax.broadcasted_iota(jnp.int32, mm.shape, 1)
    row = jax.lax.broadcasted_iota(jnp.int32, mm.shape, 0)
    a = lane // _LANES
    ch = lane - a * _LANES
    gy = (row // _G).astype(jnp.float32)
    gx = (row % _G).astype(jnp.float32)
    sig = jax.nn.sigmoid(mm)
    ex = jnp.exp(mm)
    aw = jnp.where(a == 0, _AW[0], jnp.where(a == 1, _AW[1], _AW[2]))
    ah = jnp.where(a == 0, _AH[0], jnp.where(a == 1, _AH[1], _AH[2]))
    out = jnp.where(ch == 0, (sig + gx) * _STRIDE,
          jnp.where(ch == 1, (sig + gy) * _STRIDE,
          jnp.where(ch == 2, ex * aw,
          jnp.where(ch == 3, ex * ah, sig))))
    for i in range(_NA):
        o_ref[0, i * _G * _G:(i + 1) * _G * _G, :] = (
            out[:, i * _LANES:i * _LANES + _NC])


def kernel(xin, W, b):
    B, C, G, _ = xin.shape
    GG = G * G
    x3 = xin.reshape(B, C, GG)
    w2 = W.reshape(_NA, _NC, C)
    w2 = jnp.pad(w2, ((0, 0), (0, _LANES - _NC), (0, 0)))
    wp = w2.reshape(_NA * _LANES, C)
    bp = jnp.pad(b.reshape(_NA, _NC), ((0, 0), (0, _LANES - _NC)))
    bp = bp.reshape(1, _NA * _LANES)
    out = pl.pallas_call(
        _yolo_body,
        grid=(B,),
        in_specs=[
            pl.BlockSpec((1, C, GG), lambda i: (i, 0, 0)),
            pl.BlockSpec((_NA * _LANES, C), lambda i: (0, 0)),
            pl.BlockSpec((1, _NA * _LANES), lambda i: (0, 0)),
        ],
        out_specs=pl.BlockSpec((1, _NA * GG, _NC), lambda i: (i, 0, 0)),
        out_shape=jax.ShapeDtypeStruct((B, _NA * GG, _NC), jnp.float32),
    )(x3, wp, bp)
    return out


# speedup vs baseline: 3.9011x; 2.0497x over previous
import jax
import jax.numpy as jnp
from jax.experimental import pallas as pl
from jax.experimental.pallas import tpu as pltpu

_STRIDE = 32.0
_G = 19
_NA = 3
_NC = 85
_LANES = 128
_AW = (116.0, 156.0, 373.0)
_AH = (90.0, 198.0, 326.0)


def _yolo_body(x_hbm, w_ref, b_ref, o_hbm, x_vmem, o_vmem, in_sem, out_sem):
    i = pl.program_id(0)
    nb = pl.num_programs(0)
    slot = jax.lax.rem(i, 2)

    def copy_in(b, s):
        return pltpu.make_async_copy(
            x_hbm.at[:, b, :], x_vmem.at[s], in_sem.at[s])

    def copy_out(b, s):
        return pltpu.make_async_copy(
            o_vmem.at[s], o_hbm.at[:, b, :], out_sem.at[s])

    @pl.when(i == 0)
    def _():
        copy_in(0, 0).start()

    @pl.when(i + 1 < nb)
    def _():
        copy_in(i + 1, 1 - slot).start()

    copy_in(i, slot).wait()
    x = x_vmem[slot]
    w = w_ref[...]
    mm = jax.lax.dot_general(
        x, w, (((1,), (0,)), ((), ())),
        preferred_element_type=jnp.float32,
        precision=jax.lax.Precision.DEFAULT)
    mm = mm + b_ref[...]
    lane = jax.lax.broadcasted_iota(jnp.int32, mm.shape, 1)
    row = jax.lax.broadcasted_iota(jnp.int32, mm.shape, 0)
    a = lane // _LANES
    ch = lane - a * _LANES
    gy = (row // _G).astype(jnp.float32)
    gx = (row % _G).astype(jnp.float32)
    sig = jax.nn.sigmoid(mm)
    ex = jnp.exp(mm)
    aw = jnp.where(a == 0, _AW[0], jnp.where(a == 1, _AW[1], _AW[2]))
    ah = jnp.where(a == 0, _AH[0], jnp.where(a == 1, _AH[1], _AH[2]))
    out = jnp.where(ch == 0, (sig + gx) * _STRIDE,
          jnp.where(ch == 1, (sig + gy) * _STRIDE,
          jnp.where(ch == 2, ex * aw,
          jnp.where(ch == 3, ex * ah, sig))))
    t = jnp.transpose(out)

    @pl.when(i >= 2)
    def _():
        copy_out(i - 2, slot).wait()

    for k in range(_NA):
        o_vmem[slot, :, k * _G * _G:(k + 1) * _G * _G] = (
            t[k * _LANES:k * _LANES + _NC, :])
    copy_out(i, slot).start()

    @pl.when(i == nb - 1)
    def _():
        @pl.when(nb >= 2)
        def _():
            copy_out(i - 1, 1 - slot).wait()
        copy_out(i, slot).wait()


def kernel(xin, W, b):
    B, C, G, _ = xin.shape
    GG = G * G
    xp = xin.transpose(2, 3, 0, 1).reshape(GG, B, C)
    w2 = W.reshape(_NA, _NC, C)
    w2 = jnp.pad(w2, ((0, 0), (0, _LANES - _NC), (0, 0)))
    wp = w2.reshape(_NA * _LANES, C).T
    bp = jnp.pad(b.reshape(_NA, _NC), ((0, 0), (0, _LANES - _NC)))
    bp = bp.reshape(1, _NA * _LANES)
    out = pl.pallas_call(
        _yolo_body,
        grid=(B,),
        in_specs=[
            pl.BlockSpec(memory_space=pltpu.MemorySpace.HBM),
            pl.BlockSpec((C, _NA * _LANES), lambda i: (0, 0)),
            pl.BlockSpec((1, _NA * _LANES), lambda i: (0, 0)),
        ],
        out_specs=pl.BlockSpec(memory_space=pltpu.MemorySpace.HBM),
        out_shape=jax.ShapeDtypeStruct((_NC, B, _NA * GG), jnp.float32),
        scratch_shapes=[
            pltpu.VMEM((2, GG, C), jnp.float32),
            pltpu.VMEM((2, _NC, _NA * GG), jnp.float32),
            pltpu.SemaphoreType.DMA((2,)),
            pltpu.SemaphoreType.DMA((2,)),
        ],
    )(xp, wp, bp)
    return out.transpose(1, 2, 0)


# speedup vs baseline: 4.3600x; 1.1176x over previous
import jax
import jax.numpy as jnp
from jax.experimental import pallas as pl
from jax.experimental.pallas import tpu as pltpu

_STRIDE = 32.0
_G = 19
_NA = 3
_NC = 85
_LANES = 128
_AW = (116.0, 156.0, 373.0)
_AH = (90.0, 198.0, 326.0)


_NSLOT = 3
_CH = 512


def _yolo_body(x_hbm, w_ref, b_ref, o_hbm, x_vmem, o_vmem, in_sem, out_sem):
    i = pl.program_id(0)
    nb = pl.num_programs(0)
    slot = jax.lax.rem(i, _NSLOT)

    def copy_in(b, s, h):
        return pltpu.make_async_copy(
            x_hbm.at[:, b, h * _CH:(h + 1) * _CH],
            x_vmem.at[s, :, h * _CH:(h + 1) * _CH],
            in_sem.at[s, h])

    def copy_out(b, s):
        return pltpu.make_async_copy(
            o_vmem.at[s], o_hbm.at[:, b, :], out_sem.at[s])

    @pl.when(i == 0)
    def _():
        for h in range(2):
            copy_in(0, 0, h).start()
            copy_in(1, 1, h).start()

    @pl.when(i + 2 < nb)
    def _():
        for h in range(2):
            copy_in(i + 2, jax.lax.rem(i + 2, _NSLOT), h).start()

    for h in range(2):
        copy_in(i, slot, h).wait()
    x = x_vmem[slot]
    w = w_ref[...]
    mm = jax.lax.dot_general(
        x, w, (((1,), (0,)), ((), ())),
        preferred_element_type=jnp.float32,
        precision=jax.lax.Precision.DEFAULT)
    mm = mm + b_ref[...]
    lane = jax.lax.broadcasted_iota(jnp.int32, mm.shape, 1)
    row = jax.lax.broadcasted_iota(jnp.int32, mm.shape, 0)
    a = lane // _LANES
    ch = lane - a * _LANES
    gy = (row // _G).astype(jnp.float32)
    gx = (row % _G).astype(jnp.float32)
    sig = jax.nn.sigmoid(mm)
    ex = jnp.exp(mm)
    aw = jnp.where(a == 0, _AW[0], jnp.where(a == 1, _AW[1], _AW[2]))
    ah = jnp.where(a == 0, _AH[0], jnp.where(a == 1, _AH[1], _AH[2]))
    out = jnp.where(ch == 0, (sig + gx) * _STRIDE,
          jnp.where(ch == 1, (sig + gy) * _STRIDE,
          jnp.where(ch == 2, ex * aw,
          jnp.where(ch == 3, ex * ah, sig))))
    t = jnp.transpose(out)

    @pl.when(i >= _NSLOT)
    def _():
        copy_out(i - _NSLOT, slot).wait()

    for k in range(_NA):
        o_vmem[slot, :, k * _G * _G:(k + 1) * _G * _G] = (
            t[k * _LANES:k * _LANES + _NC, :])
    copy_out(i, slot).start()

    @pl.when(i == nb - 1)
    def _():
        for d in range(_NSLOT - 1):
            copy_out(i - 1 - d, jax.lax.rem(i - 1 - d, _NSLOT)).wait()
        copy_out(i, slot).wait()


def kernel(xin, W, b):
    B, C, G, _ = xin.shape
    GG = G * G
    xp = xin.transpose(2, 3, 0, 1).reshape(GG, B, C)
    w2 = W.reshape(_NA, _NC, C)
    w2 = jnp.pad(w2, ((0, 0), (0, _LANES - _NC), (0, 0)))
    wp = w2.reshape(_NA * _LANES, C).T
    bp = jnp.pad(b.reshape(_NA, _NC), ((0, 0), (0, _LANES - _NC)))
    bp = bp.reshape(1, _NA * _LANES)
    out = pl.pallas_call(
        _yolo_body,
        grid=(B,),
        in_specs=[
            pl.BlockSpec(memory_space=pltpu.MemorySpace.HBM),
            pl.BlockSpec((C, _NA * _LANES), lambda i: (0, 0)),
            pl.BlockSpec((1, _NA * _LANES), lambda i: (0, 0)),
        ],
        out_specs=pl.BlockSpec(memory_space=pltpu.MemorySpace.HBM),
        out_shape=jax.ShapeDtypeStruct((_NC, B, _NA * GG), jnp.float32),
        scratch_shapes=[
            pltpu.VMEM((_NSLOT, GG, C), jnp.float32),
            pltpu.VMEM((_NSLOT, _NC, _NA * GG), jnp.float32),
            pltpu.SemaphoreType.DMA((_NSLOT, 2)),
            pltpu.SemaphoreType.DMA((_NSLOT,)),
        ],
    )(xp, wp, bp)
    return out.transpose(1, 2, 0)


# speedup vs baseline: 5.4982x; 1.2611x over previous
import jax
import jax.numpy as jnp
from jax.experimental import pallas as pl
from jax.experimental.pallas import tpu as pltpu

_STRIDE = 32.0
_G = 19
_NA = 3
_NC = 85
_LANES = 128
_AW = (116.0, 156.0, 373.0)
_AH = (90.0, 198.0, 326.0)


_NSLOT = 3
_CH = 512


def _yolo_body(x_hbm, w_ref, b_ref, o_hbm, x_vmem, o_vmem, in_sem, out_sem):
    i = pl.program_id(0)
    nb = pl.num_programs(0)
    slot = jax.lax.rem(i, _NSLOT)

    def copy_in(b, s, h):
        return pltpu.make_async_copy(
            x_hbm.at[:, b, h * _CH:(h + 1) * _CH],
            x_vmem.at[s, :, h * _CH:(h + 1) * _CH],
            in_sem.at[s, h])

    def copy_out(b, s):
        return pltpu.make_async_copy(
            o_vmem.at[s], o_hbm.at[:, b, :], out_sem.at[s])

    @pl.when(i == 0)
    def _():
        for h in range(2):
            copy_in(0, 0, h).start()
            copy_in(1, 1, h).start()

    @pl.when(i + 2 < nb)
    def _():
        for h in range(2):
            copy_in(i + 2, jax.lax.rem(i + 2, _NSLOT), h).start()

    for h in range(2):
        copy_in(i, slot, h).wait()
    x = x_vmem[slot]
    w = w_ref[...]
    mm = jax.lax.dot_general(
        w, x, (((1,), (1,)), ((), ())),
        preferred_element_type=jnp.float32,
        precision=jax.lax.Precision.DEFAULT)
    mm = mm + b_ref[...]

    @pl.when(i >= _NSLOT)
    def _():
        copy_out(i - _NSLOT, slot).wait()

    lane = jax.lax.broadcasted_iota(jnp.int32, (_NC, _G * _G), 1)
    row = jax.lax.broadcasted_iota(jnp.int32, (_NC, _G * _G), 0)
    gy = (lane // _G).astype(jnp.float32)
    gx = (lane % _G).astype(jnp.float32)
    for k in range(_NA):
        s = mm[k * _LANES:k * _LANES + _NC, :]
        sig = jax.nn.sigmoid(s)
        bw = jnp.broadcast_to(jnp.exp(s[2:3, :]) * _AW[k], s.shape)
        bh = jnp.broadcast_to(jnp.exp(s[3:4, :]) * _AH[k], s.shape)
        val = jnp.where(row == 0, (sig + gx) * _STRIDE,
              jnp.where(row == 1, (sig + gy) * _STRIDE,
              jnp.where(row == 2, bw,
              jnp.where(row == 3, bh, sig))))
        o_vmem[slot, :, k * _G * _G:(k + 1) * _G * _G] = val
    copy_out(i, slot).start()

    @pl.when(i == nb - 1)
    def _():
        for d in range(_NSLOT - 1):
            copy_out(i - 1 - d, jax.lax.rem(i - 1 - d, _NSLOT)).wait()
        copy_out(i, slot).wait()


def kernel(xin, W, b):
    B, C, G, _ = xin.shape
    GG = G * G
    xp = xin.transpose(2, 3, 0, 1).reshape(GG, B, C)
    w2 = W.reshape(_NA, _NC, C)
    w2 = jnp.pad(w2, ((0, 0), (0, _LANES - _NC), (0, 0)))
    wp = w2.reshape(_NA * _LANES, C)
    bp = jnp.pad(b.reshape(_NA, _NC), ((0, 0), (0, _LANES - _NC)))
    bp = bp.reshape(_NA * _LANES, 1)
    out = pl.pallas_call(
        _yolo_body,
        grid=(B,),
        in_specs=[
            pl.BlockSpec(memory_space=pltpu.MemorySpace.HBM),
            pl.BlockSpec((_NA * _LANES, C), lambda i: (0, 0)),
            pl.BlockSpec((_NA * _LANES, 1), lambda i: (0, 0)),
        ],
        out_specs=pl.BlockSpec(memory_space=pltpu.MemorySpace.HBM),
        out_shape=jax.ShapeDtypeStruct((_NC, B, _NA * GG), jnp.float32),
        scratch_shapes=[
            pltpu.VMEM((_NSLOT, GG, C), jnp.float32),
            pltpu.VMEM((_NSLOT, _NC, _NA * GG), jnp.float32),
            pltpu.SemaphoreType.DMA((_NSLOT, 2)),
            pltpu.SemaphoreType.DMA((_NSLOT,)),
        ],
    )(xp, wp, bp)
    return out.transpose(1, 2, 0)


# speedup vs baseline: 5.5211x; 1.0042x over previous
import jax
import jax.numpy as jnp
from jax.experimental import pallas as pl
from jax.experimental.pallas import tpu as pltpu

_STRIDE = 32.0
_G = 19
_NA = 3
_NC = 85
_LANES = 128
_AW = (116.0, 156.0, 373.0)
_AH = (90.0, 198.0, 326.0)


_NSLOT = 3
_CH = 512


def _yolo_body(x_hbm, w_ref, b_ref, o_hbm, x_vmem, o_vmem, in_sem, out_sem):
    i = pl.program_id(0)
    nb = pl.num_programs(0)
    slot = jax.lax.rem(i, _NSLOT)

    def copy_in(b, s, h):
        return pltpu.make_async_copy(
            x_hbm.at[:, b, h * _CH:(h + 1) * _CH],
            x_vmem.at[s, :, h * _CH:(h + 1) * _CH],
            in_sem.at[s, h])

    def copy_out(b, s):
        return pltpu.make_async_copy(
            o_vmem.at[s], o_hbm.at[:, b, :], out_sem.at[s])

    @pl.when(i == 0)
    def _():
        for h in range(2):
            copy_in(0, 0, h).start()
            copy_in(1, 1, h).start()

    @pl.when(i + 2 < nb)
    def _():
        for h in range(2):
            copy_in(i + 2, jax.lax.rem(i + 2, _NSLOT), h).start()

    for h in range(2):
        copy_in(i, slot, h).wait()
    x = x_vmem[slot].astype(jnp.bfloat16)
    w = w_ref[...]
    mm = jax.lax.dot_general(
        w, x, (((1,), (1,)), ((), ())),
        preferred_element_type=jnp.float32)
    mm = mm + b_ref[...]

    @pl.when(i >= _NSLOT)
    def _():
        copy_out(i - _NSLOT, slot).wait()

    lane = jax.lax.broadcasted_iota(jnp.int32, (_NC, _G * _G), 1)
    row = jax.lax.broadcasted_iota(jnp.int32, (_NC, _G * _G), 0)
    gy = (lane // _G).astype(jnp.float32)
    gx = (lane % _G).astype(jnp.float32)
    for k in range(_NA):
        s = mm[k * _LANES:k * _LANES + _NC, :]
        sig = jax.nn.sigmoid(s)
        bw = jnp.broadcast_to(jnp.exp(s[2:3, :]) * _AW[k], s.shape)
        bh = jnp.broadcast_to(jnp.exp(s[3:4, :]) * _AH[k], s.shape)
        val = jnp.where(row == 0, (sig + gx) * _STRIDE,
              jnp.where(row == 1, (sig + gy) * _STRIDE,
              jnp.where(row == 2, bw,
              jnp.where(row == 3, bh, sig))))
        o_vmem[slot, :, k * _G * _G:(k + 1) * _G * _G] = val
    copy_out(i, slot).start()

    @pl.when(i == nb - 1)
    def _():
        for d in range(_NSLOT - 1):
            copy_out(i - 1 - d, jax.lax.rem(i - 1 - d, _NSLOT)).wait()
        copy_out(i, slot).wait()


def kernel(xin, W, b):
    B, C, G, _ = xin.shape
    GG = G * G
    xp = xin.transpose(2, 3, 0, 1).reshape(GG, B, C)
    w2 = W.reshape(_NA, _NC, C)
    w2 = jnp.pad(w2, ((0, 0), (0, _LANES - _NC), (0, 0)))
    wp = w2.reshape(_NA * _LANES, C).astype(jnp.bfloat16)
    bp = jnp.pad(b.reshape(_NA, _NC), ((0, 0), (0, _LANES - _NC)))
    bp = bp.reshape(_NA * _LANES, 1)
    out = pl.pallas_call(
        _yolo_body,
        grid=(B,),
        in_specs=[
            pl.BlockSpec(memory_space=pltpu.MemorySpace.HBM),
            pl.BlockSpec((_NA * _LANES, C), lambda i: (0, 0)),
            pl.BlockSpec((_NA * _LANES, 1), lambda i: (0, 0)),
        ],
        out_specs=pl.BlockSpec(memory_space=pltpu.MemorySpace.HBM),
        out_shape=jax.ShapeDtypeStruct((_NC, B, _NA * GG), jnp.float32),
        scratch_shapes=[
            pltpu.VMEM((_NSLOT, GG, C), jnp.float32),
            pltpu.VMEM((_NSLOT, _NC, _NA * GG), jnp.float32),
            pltpu.SemaphoreType.DMA((_NSLOT, 2)),
            pltpu.SemaphoreType.DMA((_NSLOT,)),
        ],
    )(xp, wp, bp)
    return out.transpose(1, 2, 0)
